# baseline (speedup 1.0000x reference)
import functools

import jax
import jax.numpy as jnp
from jax import lax
from jax.experimental import pallas as pl
from jax.experimental.pallas import tpu as pltpu
from jax.experimental.pallas import tpu_sc as plsc

_B = 1024
_DIM = 128
_DIM_P = 64
_C = 100
_T = 4
_N = 100000
_TAU = 3.0
_ALPHA = 1.0 - 0.9 * 20.0 / 100.0
_CUR_TEA = 3
_TEA_IDX = 3
_ROWS = _T * _N
_G = _CUR_TEA * _B

_NC = 2
_NS = 16
_NW = _NC * _NS
_GPW = _G // _NW
_SPW = _B // _NW

_HIGHEST = lax.Precision.HIGHEST


_RB = 10000


def _copy_body(ks, kd):
    kd[...] = ks[...]


_copy_keys = pl.pallas_call(
    _copy_body,
    grid=(_ROWS // _RB,),
    in_specs=[pl.BlockSpec((_RB, _DIM), lambda i: (i, 0))],
    out_specs=pl.BlockSpec((_RB, _DIM), lambda i: (i, 0)),
    out_shape=jax.ShapeDtypeStruct((_ROWS, _DIM), jnp.float32),
)

_VCH = 400
_NCHUNKS = _ROWS // _VCH


@functools.lru_cache(maxsize=None)
def _sc_kernels():
    vmesh = plsc.VectorSubcoreMesh(core_axis_name="c", subcore_axis_name="s",
                                   num_cores=_NC, num_subcores=_NS)

    @functools.partial(
        pl.kernel,
        out_type=jax.ShapeDtypeStruct((_G, _DIM), jnp.float32),
        mesh=vmesh,
        scratch_types=[pltpu.VMEM((_GPW,), jnp.int32),
                       pltpu.VMEM((_GPW, _DIM), jnp.float32),
                       pltpu.SemaphoreType.DMA],
    )
    def _tec_kgather(kflat, gidx, tk_out, gi_v, krows, s1):
        wid = lax.axis_index("s") * _NC + lax.axis_index("c")
        base = wid * _GPW
        pltpu.sync_copy(gidx.at[pl.ds(base, _GPW)], gi_v)
        pltpu.async_copy(kflat.at[gi_v], krows, s1).wait()
        pltpu.sync_copy(krows, tk_out.at[pl.ds(base, _GPW)])

    @functools.partial(
        pl.kernel,
        out_type=(),
        mesh=vmesh,
        scratch_types=[pltpu.VMEM((_SPW,), jnp.int32),
                       pltpu.VMEM((_SPW, _DIM), jnp.float32),
                       pltpu.SemaphoreType.DMA],
    )
    def _tec_kscatter(kbank, sidx, qrows, si_v, krows, s1):
        wid = lax.axis_index("s") * _NC + lax.axis_index("c")
        base = wid * _SPW
        pltpu.sync_copy(sidx.at[pl.ds(base, _SPW)], si_v)
        pltpu.sync_copy(qrows.at[pl.ds(base, _SPW)], krows)
        pltpu.async_copy(krows, kbank.at[si_v], s1).wait()

    @functools.partial(
        pl.kernel,
        out_type=jax.ShapeDtypeStruct((_ROWS, _C), jnp.float32),
        mesh=vmesh,
        scratch_types=[pltpu.VMEM((_VCH, _C), jnp.float32),
                       pltpu.VMEM((_VCH, _C), jnp.float32),
                       pltpu.SemaphoreType.DMA,
                       pltpu.SemaphoreType.DMA],
    )
    def _tec_vcopy(vsrc, vdst, buf0, buf1, lsem, ssem):
        wid = lax.axis_index("s") * _NC + lax.axis_index("c")
        nch = 31 + jnp.where(wid < _NCHUNKS - 31 * _NW, 1, 0)

        def body(j, _):
            base = (wid + _NW * j) * _VCH

            @pl.when(j % 2 == 0)
            def _():
                @pl.when(j >= 2)
                def _():
                    b2 = (wid + _NW * (j - 2)) * _VCH
                    pltpu.make_async_copy(
                        buf0, vdst.at[pl.ds(b2, _VCH)], ssem).wait()
                pltpu.async_copy(
                    vsrc.at[pl.ds(base, _VCH)], buf0, lsem).wait()
                pltpu.make_async_copy(
                    buf0, vdst.at[pl.ds(base, _VCH)], ssem).start()

            @pl.when(j % 2 == 1)
            def _():
                @pl.when(j >= 2)
                def _():
                    b2 = (wid + _NW * (j - 2)) * _VCH
                    pltpu.make_async_copy(
                        buf1, vdst.at[pl.ds(b2, _VCH)], ssem).wait()
                pltpu.async_copy(
                    vsrc.at[pl.ds(base, _VCH)], buf1, lsem).wait()
                pltpu.make_async_copy(
                    buf1, vdst.at[pl.ds(base, _VCH)], ssem).start()

            return 0

        lax.fori_loop(0, nch, body, 0)
        pltpu.make_async_copy(buf0, vdst.at[pl.ds(0, _VCH)], ssem).wait()
        pltpu.make_async_copy(buf1, vdst.at[pl.ds(0, _VCH)], ssem).wait()

    return _tec_kgather, _tec_kscatter, _tec_vcopy


def _compute_body(gidx_s, sidx_s, idxc_r, idxr_r, y_r,
                  q_r, l_r, tk_r, wq_r, bq_r, wk_r, bk_r, nv,
                  l1_r, l2_r, ft_r, qres_r, nv_out,
                  tvbuf, lbuf, gsem, ssem):
    del nv_out
    f32 = jnp.float32
    query = q_r[...]
    logits = l_r[...]

    def _g(b, _):
        r = gidx_s[b]
        pltpu.make_async_copy(
            nv.at[pl.ds(r, 1)], tvbuf.at[pl.ds(b, 1)], gsem).start()
        return 0

    lax.fori_loop(0, _G, _g, 0, unroll=8)

    q = lax.dot_general(query, wq_r[...], (((1,), (1,)), ((), ())),
                        preferred_element_type=f32) + bq_r[...]
    v = lax.dot_general(q, wk_r[...], (((1,), (0,)), ((), ())),
                        preferred_element_type=f32)
    qbk = lax.dot_general(q, bk_r[...], (((1,), (0,)), ((), ())),
                          preferred_element_type=f32)

    es = []
    for t in range(_CUR_TEA):
        kt = tk_r[pl.ds(t * _B, _B), :]
        es.append(jnp.sum(v * kt, axis=1, keepdims=True) + qbk)
    m = jnp.maximum(jnp.maximum(es[0], es[1]), es[2])
    ws = [jnp.exp(e - m) for e in es]
    sden = ws[0] + ws[1] + ws[2]

    pltpu.make_async_copy(nv.at[pl.ds(0, _G)], tvbuf, gsem).wait()
    ft = (ws[0] / sden) * tvbuf[pl.ds(0, _B), :]
    ft = ft + (ws[1] / sden) * tvbuf[pl.ds(_B, _B), :]
    ft = ft + (ws[2] / sden) * tvbuf[pl.ds(2 * _B, _B), :]

    z = ft * (1.0 / _TAU)
    zm = jnp.max(z, axis=1, keepdims=True)
    ez = jnp.exp(z - zm)
    p = ez / jnp.sum(ez, axis=1, keepdims=True)
    ft_r[...] = p

    lmax = jnp.max(logits, axis=1, keepdims=True)
    lse = jnp.log(jnp.sum(jnp.exp(logits - lmax), axis=1, keepdims=True)) + lmax
    cls_iota = lax.broadcasted_iota(jnp.int32, (_B, _C), 1)
    oh_y = (cls_iota == y_r[...]).astype(f32)
    picked = jnp.sum(logits * oh_y, axis=1, keepdims=True)
    ce_col = lse - picked
    l1_r[...] = _ALPHA * (1.0 / _B) * jnp.sum(ce_col, axis=0, keepdims=True)

    zs = logits * (1.0 / _TAU)
    zsm = jnp.max(zs, axis=1, keepdims=True)
    lse_s = jnp.log(jnp.sum(jnp.exp(zs - zsm), axis=1, keepdims=True)) + zsm
    logp_s = zs - lse_s
    kl_rows = jnp.sum(p * (jnp.log(p + 1e-12) - logp_s), axis=1, keepdims=True)
    l2_r[...] = ((1.0 - _ALPHA) * _TAU * _TAU / _B) * jnp.sum(
        kl_rows, axis=0, keepdims=True)

    ch = 512
    jiota = lax.broadcasted_iota(jnp.int32, (ch, _B), 1)
    for c in range(_B // ch):
        rows = pl.ds(c * ch, ch)
        idc = idxc_r[rows, :]
        eq = idc == idxr_r[...]
        jsel = jnp.where(eq, jiota, -1)
        w = jnp.max(jsel, axis=1, keepdims=True)
        oh = (jiota == w).astype(f32)
        qres_r[rows, :] = lax.dot_general(
            oh, query, (((1,), (0,)), ((), ())),
            preferred_element_type=f32, precision=_HIGHEST)
        lbuf[rows, :] = lax.dot_general(
            oh, logits, (((1,), (0,)), ((), ())),
            preferred_element_type=f32, precision=_HIGHEST)

    def _s(j, _):
        r = sidx_s[j]
        pltpu.make_async_copy(
            lbuf.at[pl.ds(j, 1)], nv.at[pl.ds(r, 1)], ssem).start()
        return 0

    lax.fori_loop(0, _B, _s, 0, unroll=8)
    pltpu.make_async_copy(lbuf, nv.at[pl.ds(0, _B)], ssem).wait()


_compute = pl.pallas_call(
    _compute_body,
    in_specs=[pl.BlockSpec(memory_space=pltpu.SMEM),
              pl.BlockSpec(memory_space=pltpu.SMEM),
              pl.BlockSpec((_B, 1), lambda: (0, 0)),
              pl.BlockSpec((1, _B), lambda: (0, 0)),
              pl.BlockSpec((_B, 1), lambda: (0, 0)),
              pl.BlockSpec((_B, _DIM), lambda: (0, 0)),
              pl.BlockSpec((_B, _C), lambda: (0, 0)),
              pl.BlockSpec((_G, _DIM), lambda: (0, 0)),
              pl.BlockSpec((_DIM_P, _DIM), lambda: (0, 0)),
              pl.BlockSpec((1, _DIM_P), lambda: (0, 0)),
              pl.BlockSpec((_DIM_P, _DIM), lambda: (0, 0)),
              pl.BlockSpec((_DIM_P, 1), lambda: (0, 0)),
              pl.BlockSpec(memory_space=pltpu.MemorySpace.HBM)],
    out_specs=[pl.BlockSpec((1, 1), lambda: (0, 0)),
               pl.BlockSpec((1, 1), lambda: (0, 0)),
               pl.BlockSpec((_B, _C), lambda: (0, 0)),
               pl.BlockSpec((_B, _DIM), lambda: (0, 0)),
               pl.BlockSpec(memory_space=pltpu.MemorySpace.HBM)],
    out_shape=[jax.ShapeDtypeStruct((1, 1), jnp.float32),
               jax.ShapeDtypeStruct((1, 1), jnp.float32),
               jax.ShapeDtypeStruct((_B, _C), jnp.float32),
               jax.ShapeDtypeStruct((_B, _DIM), jnp.float32),
               jax.ShapeDtypeStruct((_ROWS, _C), jnp.float32)],
    input_output_aliases={12: 4},
    scratch_shapes=[pltpu.VMEM((_G, _C), jnp.float32),
                    pltpu.VMEM((_B, _C), jnp.float32),
                    pltpu.SemaphoreType.DMA,
                    pltpu.SemaphoreType.DMA],
)


def kernel(batch_idx, query, logits, y_true, keys_mem, values_mem,
           Wq, bq, Wk, bk):
    idx = batch_idx.astype(jnp.int32)
    kflat = keys_mem.reshape(_ROWS, _DIM)
    vflat = values_mem.reshape(_ROWS, _C)

    gidx = jnp.concatenate([idx, idx + _N, idx + 2 * _N])
    sidx = idx + _TEA_IDX * _N

    kg, ksc, vcp = _sc_kernels()
    ck = _copy_keys(kflat)
    cv = vcp(vflat)
    tk = kg(kflat, gidx)

    loss1, loss2, ft, qres, nv = _compute(
        gidx, sidx,
        idx.reshape(_B, 1), idx.reshape(1, _B), y_true.reshape(_B, 1),
        query, logits, tk,
        Wq, bq.reshape(1, _DIM_P), Wk, bk.reshape(_DIM_P, 1), cv)

    kref = jax.new_ref(ck)
    ksc(kref, sidx, qres)

    new_keys = kref[...].reshape(_T, _N, _DIM)
    new_values = nv.reshape(_T, _N, _C)
    return (loss1.reshape(()), loss2.reshape(()), ft, new_keys, new_values)


# speedup vs baseline: 1.0222x; 1.0222x over previous
import functools

import jax
import jax.numpy as jnp
from jax import lax
from jax.experimental import pallas as pl
from jax.experimental.pallas import tpu as pltpu
from jax.experimental.pallas import tpu_sc as plsc

_B = 1024
_DIM = 128
_DIM_P = 64
_C = 100
_T = 4
_N = 100000
_TAU = 3.0
_ALPHA = 1.0 - 0.9 * 20.0 / 100.0
_CUR_TEA = 3
_TEA_IDX = 3
_ROWS = _T * _N
_G = _CUR_TEA * _B

_NC = 2
_NS = 16
_NW = _NC * _NS
_GPW = _G // _NW
_SPW = _B // _NW

_HIGHEST = lax.Precision.HIGHEST


_RB = 10000


def _copy_body(vs, vd):
    vd[...] = vs[...]


_copy_vals = pl.pallas_call(
    _copy_body,
    grid=(_ROWS // _RB,),
    in_specs=[pl.BlockSpec((_RB, _C), lambda i: (i, 0))],
    out_specs=pl.BlockSpec((_RB, _C), lambda i: (i, 0)),
    out_shape=jax.ShapeDtypeStruct((_ROWS, _C), jnp.float32),
)

_VCH = 400
_NCHUNKS = _ROWS // _VCH


@functools.lru_cache(maxsize=None)
def _sc_kernels():
    vmesh = plsc.VectorSubcoreMesh(core_axis_name="c", subcore_axis_name="s",
                                   num_cores=_NC, num_subcores=_NS)

    @functools.partial(
        pl.kernel,
        out_type=jax.ShapeDtypeStruct((_G, _DIM), jnp.float32),
        mesh=vmesh,
        scratch_types=[pltpu.VMEM((_GPW,), jnp.int32),
                       pltpu.VMEM((_GPW, _DIM), jnp.float32),
                       pltpu.SemaphoreType.DMA],
    )
    def _tec_kgather(kflat, gidx, tk_out, gi_v, krows, s1):
        wid = lax.axis_index("s") * _NC + lax.axis_index("c")
        base = wid * _GPW
        pltpu.sync_copy(gidx.at[pl.ds(base, _GPW)], gi_v)
        pltpu.async_copy(kflat.at[gi_v], krows, s1).wait()
        pltpu.sync_copy(krows, tk_out.at[pl.ds(base, _GPW)])

    @functools.partial(
        pl.kernel,
        out_type=(),
        mesh=vmesh,
        scratch_types=[pltpu.VMEM((_SPW,), jnp.int32),
                       pltpu.VMEM((_SPW, _DIM), jnp.float32),
                       pltpu.SemaphoreType.DMA],
    )
    def _tec_kscatter(kbank, sidx, qrows, si_v, krows, s1):
        wid = lax.axis_index("s") * _NC + lax.axis_index("c")
        base = wid * _SPW
        pltpu.sync_copy(sidx.at[pl.ds(base, _SPW)], si_v)
        pltpu.sync_copy(qrows.at[pl.ds(base, _SPW)], krows)
        pltpu.async_copy(krows, kbank.at[si_v], s1).wait()

    @functools.partial(
        pl.kernel,
        out_type=jax.ShapeDtypeStruct((_ROWS, _DIM), jnp.float32),
        mesh=vmesh,
        scratch_types=[pltpu.VMEM((_VCH, _DIM), jnp.float32),
                       pltpu.VMEM((_VCH, _DIM), jnp.float32),
                       pltpu.SemaphoreType.DMA,
                       pltpu.SemaphoreType.DMA],
    )
    def _tec_kcopy(vsrc, vdst, buf0, buf1, lsem, ssem):
        wid = lax.axis_index("s") * _NC + lax.axis_index("c")
        nch = 31 + jnp.where(wid < _NCHUNKS - 31 * _NW, 1, 0)

        def body(j, _):
            base = (wid + _NW * j) * _VCH

            @pl.when(j % 2 == 0)
            def _():
                @pl.when(j >= 2)
                def _():
                    b2 = (wid + _NW * (j - 2)) * _VCH
                    pltpu.make_async_copy(
                        buf0, vdst.at[pl.ds(b2, _VCH)], ssem).wait()
                pltpu.async_copy(
                    vsrc.at[pl.ds(base, _VCH)], buf0, lsem).wait()
                pltpu.make_async_copy(
                    buf0, vdst.at[pl.ds(base, _VCH)], ssem).start()

            @pl.when(j % 2 == 1)
            def _():
                @pl.when(j >= 2)
                def _():
                    b2 = (wid + _NW * (j - 2)) * _VCH
                    pltpu.make_async_copy(
                        buf1, vdst.at[pl.ds(b2, _VCH)], ssem).wait()
                pltpu.async_copy(
                    vsrc.at[pl.ds(base, _VCH)], buf1, lsem).wait()
                pltpu.make_async_copy(
                    buf1, vdst.at[pl.ds(base, _VCH)], ssem).start()

            return 0

        lax.fori_loop(0, nch, body, 0)
        pltpu.make_async_copy(buf0, vdst.at[pl.ds(0, _VCH)], ssem).wait()
        pltpu.make_async_copy(buf1, vdst.at[pl.ds(0, _VCH)], ssem).wait()

    return _tec_kgather, _tec_kscatter, _tec_kcopy


def _compute_body(gidx_s, sidx_s, idxc_r, idxr_r, y_r,
                  q_r, l_r, tk_r, wq_r, bq_r, wk_r, bk_r, nv,
                  l1_r, l2_r, ft_r, qres_r, nv_out,
                  tvbuf, lbuf, gsem, ssem):
    del nv_out
    f32 = jnp.float32
    query = q_r[...]
    logits = l_r[...]

    def _g(b, _):
        r = gidx_s[b]
        pltpu.make_async_copy(
            nv.at[pl.ds(r, 1)], tvbuf.at[pl.ds(b, 1)], gsem).start()
        return 0

    lax.fori_loop(0, _G, _g, 0, unroll=8)

    q = lax.dot_general(query, wq_r[...], (((1,), (1,)), ((), ())),
                        preferred_element_type=f32) + bq_r[...]
    v = lax.dot_general(q, wk_r[...], (((1,), (0,)), ((), ())),
                        preferred_element_type=f32)
    qbk = lax.dot_general(q, bk_r[...], (((1,), (0,)), ((), ())),
                          preferred_element_type=f32)

    es = []
    for t in range(_CUR_TEA):
        kt = tk_r[pl.ds(t * _B, _B), :]
        es.append(jnp.sum(v * kt, axis=1, keepdims=True) + qbk)
    m = jnp.maximum(jnp.maximum(es[0], es[1]), es[2])
    ws = [jnp.exp(e - m) for e in es]
    sden = ws[0] + ws[1] + ws[2]

    pltpu.make_async_copy(nv.at[pl.ds(0, _G)], tvbuf, gsem).wait()
    ft = (ws[0] / sden) * tvbuf[pl.ds(0, _B), :]
    ft = ft + (ws[1] / sden) * tvbuf[pl.ds(_B, _B), :]
    ft = ft + (ws[2] / sden) * tvbuf[pl.ds(2 * _B, _B), :]

    z = ft * (1.0 / _TAU)
    zm = jnp.max(z, axis=1, keepdims=True)
    ez = jnp.exp(z - zm)
    p = ez / jnp.sum(ez, axis=1, keepdims=True)
    ft_r[...] = p

    lmax = jnp.max(logits, axis=1, keepdims=True)
    lse = jnp.log(jnp.sum(jnp.exp(logits - lmax), axis=1, keepdims=True)) + lmax
    cls_iota = lax.broadcasted_iota(jnp.int32, (_B, _C), 1)
    oh_y = (cls_iota == y_r[...]).astype(f32)
    picked = jnp.sum(logits * oh_y, axis=1, keepdims=True)
    ce_col = lse - picked
    l1_r[...] = _ALPHA * (1.0 / _B) * jnp.sum(ce_col, axis=0, keepdims=True)

    zs = logits * (1.0 / _TAU)
    zsm = jnp.max(zs, axis=1, keepdims=True)
    lse_s = jnp.log(jnp.sum(jnp.exp(zs - zsm), axis=1, keepdims=True)) + zsm
    logp_s = zs - lse_s
    kl_rows = jnp.sum(p * (jnp.log(p + 1e-12) - logp_s), axis=1, keepdims=True)
    l2_r[...] = ((1.0 - _ALPHA) * _TAU * _TAU / _B) * jnp.sum(
        kl_rows, axis=0, keepdims=True)

    ch = 512
    jiota = lax.broadcasted_iota(jnp.int32, (ch, _B), 1)
    for c in range(_B // ch):
        rows = pl.ds(c * ch, ch)
        idc = idxc_r[rows, :]
        eq = idc == idxr_r[...]
        jsel = jnp.where(eq, jiota, -1)
        w = jnp.max(jsel, axis=1, keepdims=True)
        oh = (jiota == w).astype(f32)
        qres_r[rows, :] = lax.dot_general(
            oh, query, (((1,), (0,)), ((), ())),
            preferred_element_type=f32, precision=_HIGHEST)
        lbuf[rows, :] = lax.dot_general(
            oh, logits, (((1,), (0,)), ((), ())),
            preferred_element_type=f32, precision=_HIGHEST)

    def _s(j, _):
        r = sidx_s[j]
        pltpu.make_async_copy(
            lbuf.at[pl.ds(j, 1)], nv.at[pl.ds(r, 1)], ssem).start()
        return 0

    lax.fori_loop(0, _B, _s, 0, unroll=8)
    pltpu.make_async_copy(lbuf, nv.at[pl.ds(0, _B)], ssem).wait()


_compute = pl.pallas_call(
    _compute_body,
    in_specs=[pl.BlockSpec(memory_space=pltpu.SMEM),
              pl.BlockSpec(memory_space=pltpu.SMEM),
              pl.BlockSpec((_B, 1), lambda: (0, 0)),
              pl.BlockSpec((1, _B), lambda: (0, 0)),
              pl.BlockSpec((_B, 1), lambda: (0, 0)),
              pl.BlockSpec((_B, _DIM), lambda: (0, 0)),
              pl.BlockSpec((_B, _C), lambda: (0, 0)),
              pl.BlockSpec((_G, _DIM), lambda: (0, 0)),
              pl.BlockSpec((_DIM_P, _DIM), lambda: (0, 0)),
              pl.BlockSpec((1, _DIM_P), lambda: (0, 0)),
              pl.BlockSpec((_DIM_P, _DIM), lambda: (0, 0)),
              pl.BlockSpec((_DIM_P, 1), lambda: (0, 0)),
              pl.BlockSpec(memory_space=pltpu.MemorySpace.HBM)],
    out_specs=[pl.BlockSpec((1, 1), lambda: (0, 0)),
               pl.BlockSpec((1, 1), lambda: (0, 0)),
               pl.BlockSpec((_B, _C), lambda: (0, 0)),
               pl.BlockSpec((_B, _DIM), lambda: (0, 0)),
               pl.BlockSpec(memory_space=pltpu.MemorySpace.HBM)],
    out_shape=[jax.ShapeDtypeStruct((1, 1), jnp.float32),
               jax.ShapeDtypeStruct((1, 1), jnp.float32),
               jax.ShapeDtypeStruct((_B, _C), jnp.float32),
               jax.ShapeDtypeStruct((_B, _DIM), jnp.float32),
               jax.ShapeDtypeStruct((_ROWS, _C), jnp.float32)],
    input_output_aliases={12: 4},
    scratch_shapes=[pltpu.VMEM((_G, _C), jnp.float32),
                    pltpu.VMEM((_B, _C), jnp.float32),
                    pltpu.SemaphoreType.DMA,
                    pltpu.SemaphoreType.DMA],
)


def kernel(batch_idx, query, logits, y_true, keys_mem, values_mem,
           Wq, bq, Wk, bk):
    idx = batch_idx.astype(jnp.int32)
    kflat = keys_mem.reshape(_ROWS, _DIM)
    vflat = values_mem.reshape(_ROWS, _C)

    gidx = jnp.concatenate([idx, idx + _N, idx + 2 * _N])
    sidx = idx + _TEA_IDX * _N

    kg, ksc, kcp = _sc_kernels()
    cv = _copy_vals(vflat)
    ck = kcp(kflat)
    tk = kg(kflat, gidx)

    loss1, loss2, ft, qres, nv = _compute(
        gidx, sidx,
        idx.reshape(_B, 1), idx.reshape(1, _B), y_true.reshape(_B, 1),
        query, logits, tk,
        Wq, bq.reshape(1, _DIM_P), Wk, bk.reshape(_DIM_P, 1), cv)

    kref = jax.new_ref(ck)
    ksc(kref, sidx, qres)

    new_keys = kref[...].reshape(_T, _N, _DIM)
    new_values = nv.reshape(_T, _N, _C)
    return (loss1.reshape(()), loss2.reshape(()), ft, new_keys, new_values)


# speedup vs baseline: 1.6268x; 1.5915x over previous
import functools

import jax
import jax.numpy as jnp
from jax import lax
from jax.experimental import pallas as pl
from jax.experimental.pallas import tpu as pltpu
from jax.experimental.pallas import tpu_sc as plsc

_B = 1024
_DIM = 128
_DIM_P = 64
_C = 100
_T = 4
_N = 100000
_TAU = 3.0
_ALPHA = 1.0 - 0.9 * 20.0 / 100.0
_CUR_TEA = 3
_TEA_IDX = 3
_ROWS = _T * _N
_G = _CUR_TEA * _B

_NC = 2
_NS = 16
_NW = _NC * _NS
_GPW = _G // _NW
_SPW = _B // _NW

_HIGHEST = lax.Precision.HIGHEST


_RB = 10000


def _copy_body(vs, vd):
    vd[...] = vs[...]


_copy_vals = pl.pallas_call(
    _copy_body,
    grid=(_T, _N // _RB),
    in_specs=[pl.BlockSpec((1, _RB, _C), lambda t, i: (t, i, 0))],
    out_specs=pl.BlockSpec((1, _RB, _C), lambda t, i: (t, i, 0)),
    out_shape=jax.ShapeDtypeStruct((_T, _N, _C), jnp.float32),
)

_VCH = 400
_NCHUNKS = _ROWS // _VCH


@functools.lru_cache(maxsize=None)
def _sc_kernels():
    vmesh = plsc.VectorSubcoreMesh(core_axis_name="c", subcore_axis_name="s",
                                   num_cores=_NC, num_subcores=_NS)

    @functools.partial(
        pl.kernel,
        out_type=jax.ShapeDtypeStruct((_G, _DIM), jnp.float32),
        mesh=vmesh,
        scratch_types=[pltpu.VMEM((_SPW,), jnp.int32),
                       pltpu.VMEM((_SPW, _DIM), jnp.float32),
                       pltpu.SemaphoreType.DMA],
    )
    def _tec_kgather(kmem, idx, tk_out, gi_v, krows, s1):
        wid = lax.axis_index("s") * _NC + lax.axis_index("c")
        base = wid * _SPW
        pltpu.sync_copy(idx.at[pl.ds(base, _SPW)], gi_v)
        for t in range(_CUR_TEA):
            pltpu.async_copy(kmem.at[t].at[gi_v], krows, s1).wait()
            pltpu.sync_copy(krows, tk_out.at[pl.ds(t * _B + base, _SPW)])

    @functools.partial(
        pl.kernel,
        out_type=(),
        mesh=vmesh,
        scratch_types=[pltpu.VMEM((_SPW,), jnp.int32),
                       pltpu.VMEM((_SPW, _DIM), jnp.float32),
                       pltpu.SemaphoreType.DMA],
    )
    def _tec_kscatter(kbank, idx, qrows, si_v, krows, s1):
        wid = lax.axis_index("s") * _NC + lax.axis_index("c")
        base = wid * _SPW
        pltpu.sync_copy(idx.at[pl.ds(base, _SPW)], si_v)
        pltpu.sync_copy(qrows.at[pl.ds(base, _SPW)], krows)
        pltpu.async_copy(krows, kbank.at[_TEA_IDX].at[si_v], s1).wait()

    _CPT = _N // _VCH

    @functools.partial(
        pl.kernel,
        out_type=jax.ShapeDtypeStruct((_T, _N, _DIM), jnp.float32),
        mesh=vmesh,
        scratch_types=[pltpu.VMEM((_VCH, _DIM), jnp.float32),
                       pltpu.VMEM((_VCH, _DIM), jnp.float32),
                       pltpu.SemaphoreType.DMA,
                       pltpu.SemaphoreType.DMA],
    )
    def _tec_kcopy(vsrc, vdst, buf0, buf1, lsem, ssem):
        wid = lax.axis_index("s") * _NC + lax.axis_index("c")
        nch = 31 + jnp.where(wid < _NCHUNKS - 31 * _NW, 1, 0)

        def slices(j):
            c = wid + _NW * j
            t = c // _CPT
            base = (c % _CPT) * _VCH
            return t, base

        def body(j, _):
            t, base = slices(j)

            @pl.when(j % 2 == 0)
            def _():
                @pl.when(j >= 2)
                def _():
                    t2, b2 = slices(j - 2)
                    pltpu.make_async_copy(
                        buf0, vdst.at[t2, pl.ds(b2, _VCH)], ssem).wait()
                pltpu.async_copy(
                    vsrc.at[t, pl.ds(base, _VCH)], buf0, lsem).wait()
                pltpu.make_async_copy(
                    buf0, vdst.at[t, pl.ds(base, _VCH)], ssem).start()

            @pl.when(j % 2 == 1)
            def _():
                @pl.when(j >= 2)
                def _():
                    t2, b2 = slices(j - 2)
                    pltpu.make_async_copy(
                        buf1, vdst.at[t2, pl.ds(b2, _VCH)], ssem).wait()
                pltpu.async_copy(
                    vsrc.at[t, pl.ds(base, _VCH)], buf1, lsem).wait()
                pltpu.make_async_copy(
                    buf1, vdst.at[t, pl.ds(base, _VCH)], ssem).start()

            return 0

        lax.fori_loop(0, nch, body, 0)
        pltpu.make_async_copy(buf0, vdst.at[0, pl.ds(0, _VCH)], ssem).wait()
        pltpu.make_async_copy(buf1, vdst.at[0, pl.ds(0, _VCH)], ssem).wait()

    return _tec_kgather, _tec_kscatter, _tec_kcopy


def _compute_body(idx_s, idxc_r, idxr_r, y_r,
                  q_r, l_r, tk_r, wq_r, bq_r, wk_r, bk_r, nv,
                  l1_r, l2_r, ft_r, qres_r, nv_out,
                  tvbuf, lbuf, gsem, ssem):
    del nv_out
    f32 = jnp.float32
    query = q_r[...]
    logits = l_r[...]

    for t in range(_CUR_TEA):
        def _g(b, _, t=t):
            r = idx_s[b]
            pltpu.make_async_copy(
                nv.at[t].at[pl.ds(r, 1)],
                tvbuf.at[pl.ds(t * _B + b, 1)], gsem).start()
            return 0

        lax.fori_loop(0, _B, _g, 0, unroll=8)

    q = lax.dot_general(query, wq_r[...], (((1,), (1,)), ((), ())),
                        preferred_element_type=f32) + bq_r[...]
    v = lax.dot_general(q, wk_r[...], (((1,), (0,)), ((), ())),
                        preferred_element_type=f32)
    qbk = lax.dot_general(q, bk_r[...], (((1,), (0,)), ((), ())),
                          preferred_element_type=f32)

    es = []
    for t in range(_CUR_TEA):
        kt = tk_r[pl.ds(t * _B, _B), :]
        es.append(jnp.sum(v * kt, axis=1, keepdims=True) + qbk)
    m = jnp.maximum(jnp.maximum(es[0], es[1]), es[2])
    ws = [jnp.exp(e - m) for e in es]
    sden = ws[0] + ws[1] + ws[2]

    pltpu.make_async_copy(nv.at[0].at[pl.ds(0, _G)], tvbuf, gsem).wait()
    ft = (ws[0] / sden) * tvbuf[pl.ds(0, _B), :]
    ft = ft + (ws[1] / sden) * tvbuf[pl.ds(_B, _B), :]
    ft = ft + (ws[2] / sden) * tvbuf[pl.ds(2 * _B, _B), :]

    z = ft * (1.0 / _TAU)
    zm = jnp.max(z, axis=1, keepdims=True)
    ez = jnp.exp(z - zm)
    p = ez / jnp.sum(ez, axis=1, keepdims=True)
    ft_r[...] = p

    lmax = jnp.max(logits, axis=1, keepdims=True)
    lse = jnp.log(jnp.sum(jnp.exp(logits - lmax), axis=1, keepdims=True)) + lmax
    cls_iota = lax.broadcasted_iota(jnp.int32, (_B, _C), 1)
    oh_y = (cls_iota == y_r[...]).astype(f32)
    picked = jnp.sum(logits * oh_y, axis=1, keepdims=True)
    ce_col = lse - picked
    l1_r[...] = _ALPHA * (1.0 / _B) * jnp.sum(ce_col, axis=0, keepdims=True)

    zs = logits * (1.0 / _TAU)
    zsm = jnp.max(zs, axis=1, keepdims=True)
    lse_s = jnp.log(jnp.sum(jnp.exp(zs - zsm), axis=1, keepdims=True)) + zsm
    logp_s = zs - lse_s
    kl_rows = jnp.sum(p * (jnp.log(p + 1e-12) - logp_s), axis=1, keepdims=True)
    l2_r[...] = ((1.0 - _ALPHA) * _TAU * _TAU / _B) * jnp.sum(
        kl_rows, axis=0, keepdims=True)

    ch = 512
    jiota = lax.broadcasted_iota(jnp.int32, (ch, _B), 1)
    for c in range(_B // ch):
        rows = pl.ds(c * ch, ch)
        idc = idxc_r[rows, :]
        eq = idc == idxr_r[...]
        jsel = jnp.where(eq, jiota, -1)
        w = jnp.max(jsel, axis=1, keepdims=True)
        oh = (jiota == w).astype(f32)
        qres_r[rows, :] = lax.dot_general(
            oh, query, (((1,), (0,)), ((), ())),
            preferred_element_type=f32, precision=_HIGHEST)
        lbuf[rows, :] = lax.dot_general(
            oh, logits, (((1,), (0,)), ((), ())),
            preferred_element_type=f32, precision=_HIGHEST)

    def _s(j, _):
        r = idx_s[j]
        pltpu.make_async_copy(
            lbuf.at[pl.ds(j, 1)], nv.at[_TEA_IDX].at[pl.ds(r, 1)],
            ssem).start()
        return 0

    lax.fori_loop(0, _B, _s, 0, unroll=8)
    pltpu.make_async_copy(lbuf, nv.at[_TEA_IDX].at[pl.ds(0, _B)], ssem).wait()


_compute = pl.pallas_call(
    _compute_body,
    in_specs=[pl.BlockSpec(memory_space=pltpu.SMEM),
              pl.BlockSpec((_B, 1), lambda: (0, 0)),
              pl.BlockSpec((1, _B), lambda: (0, 0)),
              pl.BlockSpec((_B, 1), lambda: (0, 0)),
              pl.BlockSpec((_B, _DIM), lambda: (0, 0)),
              pl.BlockSpec((_B, _C), lambda: (0, 0)),
              pl.BlockSpec((_G, _DIM), lambda: (0, 0)),
              pl.BlockSpec((_DIM_P, _DIM), lambda: (0, 0)),
              pl.BlockSpec((1, _DIM_P), lambda: (0, 0)),
              pl.BlockSpec((_DIM_P, _DIM), lambda: (0, 0)),
              pl.BlockSpec((_DIM_P, 1), lambda: (0, 0)),
              pl.BlockSpec(memory_space=pltpu.MemorySpace.HBM)],
    out_specs=[pl.BlockSpec((1, 1), lambda: (0, 0)),
               pl.BlockSpec((1, 1), lambda: (0, 0)),
               pl.BlockSpec((_B, _C), lambda: (0, 0)),
               pl.BlockSpec((_B, _DIM), lambda: (0, 0)),
               pl.BlockSpec(memory_space=pltpu.MemorySpace.HBM)],
    out_shape=[jax.ShapeDtypeStruct((1, 1), jnp.float32),
               jax.ShapeDtypeStruct((1, 1), jnp.float32),
               jax.ShapeDtypeStruct((_B, _C), jnp.float32),
               jax.ShapeDtypeStruct((_B, _DIM), jnp.float32),
               jax.ShapeDtypeStruct((_T, _N, _C), jnp.float32)],
    input_output_aliases={11: 4},
    scratch_shapes=[pltpu.VMEM((_G, _C), jnp.float32),
                    pltpu.VMEM((_B, _C), jnp.float32),
                    pltpu.SemaphoreType.DMA,
                    pltpu.SemaphoreType.DMA],
)


def kernel(batch_idx, query, logits, y_true, keys_mem, values_mem,
           Wq, bq, Wk, bk):
    idx = batch_idx.astype(jnp.int32)

    kg, ksc, kcp = _sc_kernels()
    cv = _copy_vals(values_mem)
    ck = kcp(keys_mem)
    tk = kg(keys_mem, idx)

    loss1, loss2, ft, qres, nv = _compute(
        idx,
        idx.reshape(_B, 1), idx.reshape(1, _B), y_true.reshape(_B, 1),
        query, logits, tk,
        Wq, bq.reshape(1, _DIM_P), Wk, bk.reshape(_DIM_P, 1), cv)

    kref = jax.new_ref(ck)
    ksc(kref, idx, qres)

    new_keys = kref[...]
    return (loss1.reshape(()), loss2.reshape(()), ft, new_keys, nv)
